# baseline (speedup 1.0000x reference)
import functools

import jax
import jax.numpy as jnp
from jax import lax
from jax.experimental import pallas as pl
from jax.experimental.pallas import tpu as pltpu
from jax.experimental.pallas import tpu_sc as plsc

VOCAB = 1000000
EMBED = 32
BATCH = 16384
HIST = 50

NUM_CORES = 2
NUM_SUBCORES = 16
NW = NUM_CORES * NUM_SUBCORES
BPW = BATCH // NW

_mesh = plsc.VectorSubcoreMesh(core_axis_name="c", subcore_axis_name="s")


@functools.partial(
    pl.kernel,
    mesh=_mesh,
    compiler_params=pltpu.CompilerParams(use_tc_tiling_on_sc=False),
    out_type=jax.ShapeDtypeStruct((BATCH, 56, 128), jnp.float32),
    scratch_types=[
        pltpu.VMEM((HIST, BPW), jnp.int32),
        pltpu.VMEM((4, BPW, EMBED), jnp.float32),
        [pltpu.SemaphoreType.DMA] * 4,
        [pltpu.SemaphoreType.DMA] * 4,
    ],
)
def _gather(idx_hbm, table_hbm, out_hbm, idx_v, rows_v, gsems, wsems):
    wid = lax.axis_index("s") * NUM_CORES + lax.axis_index("c")
    b0 = wid * BPW

    pltpu.sync_copy(idx_hbm.at[:, pl.ds(b0, BPW)], idx_v)

    def gather_start(h):
        return pltpu.async_copy(
            table_hbm.at[idx_v.at[h]], rows_v.at[h % 4], gsems[h % 4])

    def wb_copy(h):
        return pltpu.make_async_copy(
            rows_v.at[h % 4],
            out_hbm.at[pl.ds(b0, BPW), h, pl.ds(0, EMBED)], wsems[h % 4])

    pending = {0: gather_start(0), 1: gather_start(1)}
    for h in range(HIST):
        pending.pop(h).wait()
        wb_copy(h).start()
        if h >= 2:
            wb_copy(h - 2).wait()
        if h + 2 < HIST:
            pending[h + 2] = gather_start(h + 2)
    wb_copy(HIST - 2).wait()
    wb_copy(HIST - 1).wait()


def kernel(x, W):
    xi = jnp.swapaxes(x, 0, 1).astype(jnp.int32)
    idx_t = (xi // 2) * 4 + (xi % 2)
    W4 = jnp.pad(
        W.reshape(VOCAB // 2, 2 * EMBED), ((0, 0), (0, 64))
    ).reshape(2 * VOCAB, EMBED)
    out_big = _gather(idx_t, W4)
    return out_big[:, :HIST, :EMBED]


# speedup vs baseline: 1.3212x; 1.3212x over previous
import functools

import jax
import jax.numpy as jnp
from jax import lax
from jax.experimental import pallas as pl
from jax.experimental.pallas import tpu as pltpu
from jax.experimental.pallas import tpu_sc as plsc

VOCAB = 1000000
EMBED = 32
BATCH = 16384
HIST = 50

NUM_CORES = 2
NUM_SUBCORES = 16
NW = NUM_CORES * NUM_SUBCORES
BPW = BATCH // NW

_mesh = plsc.VectorSubcoreMesh(core_axis_name="c", subcore_axis_name="s")


@functools.partial(
    pl.kernel,
    mesh=_mesh,
    compiler_params=pltpu.CompilerParams(use_tc_tiling_on_sc=False),
    out_type=jax.ShapeDtypeStruct((BATCH, 56, 128), jnp.float32),
    scratch_types=[
        pltpu.VMEM((HIST, BPW), jnp.int32),
        pltpu.VMEM((4, BPW, EMBED), jnp.float32),
        [pltpu.SemaphoreType.DMA] * 4,
        [pltpu.SemaphoreType.DMA] * 4,
    ],
)
def _gather(idx_hbm, table_hbm, out_hbm, idx_v, rows_v, gsems, wsems):
    wid = lax.axis_index("s") * NUM_CORES + lax.axis_index("c")
    b0 = wid * BPW

    pltpu.sync_copy(idx_hbm.at[:, pl.ds(b0, BPW)], idx_v)

    def gather_start(h):
        return pltpu.async_copy(
            table_hbm.at[idx_v.at[h]], rows_v.at[h % 4], gsems[h % 4])

    def wb_copy(h):
        return pltpu.make_async_copy(
            rows_v.at[h % 4],
            out_hbm.at[pl.ds(b0, BPW), h, pl.ds(0, EMBED)], wsems[h % 4])

    pending = {0: gather_start(0), 1: gather_start(1)}
    for h in range(HIST):
        pending.pop(h).wait()
        wb_copy(h).start()
        if h >= 2:
            wb_copy(h - 2).wait()
        if h + 2 < HIST:
            pending[h + 2] = gather_start(h + 2)
    wb_copy(HIST - 2).wait()
    wb_copy(HIST - 1).wait()


def kernel(x, W):
    idx_t = jnp.swapaxes(x, 0, 1).astype(jnp.int32) * 4
    W4 = jnp.pad(W, ((0, 0), (0, 128 - EMBED))).reshape(4 * VOCAB, EMBED)
    out_big = _gather(idx_t, W4)
    return out_big[:, :HIST, :EMBED]
